# baseline (speedup 1.0000x reference)
import functools

import jax
import jax.numpy as jnp
from jax import lax
from jax.experimental import pallas as pl
from jax.experimental.pallas import tpu as pltpu
from jax.experimental.pallas import tpu_sc as plsc

_NC = 2
_NS = 16
_LANES = 16
_NW = _NC * _NS

_NROW = 4096
_NCOL = 4096
_ROWS_W = _NROW // _NW
_CROWS = 1
_NCHUNK = _ROWS_W // _CROWS
_RB = 8
_NTURN = _NCHUNK // _RB


def _compute_chunk(xv, ov):
    c16 = jnp.full((_LANES,), 16, jnp.int32)
    c17 = jnp.full((_LANES,), 17, jnp.int32)
    for r in range(_CROWS):
        @plsc.parallel_loop(0, _NCOL, step=_LANES, unroll=16)
        def _(j):
            x = xv[r, pl.ds(j, _LANES)]
            y = jnp.minimum(jnp.maximum(x * 4.0, -17.0), 16.0)
            i = y.astype(jnp.int32)
            f = i.astype(jnp.float32)
            b = i + jnp.where(f > y, c16, c17)
            ov[r, pl.ds(j, _LANES)] = b


@functools.partial(
    pl.kernel,
    mesh=plsc.VectorSubcoreMesh(core_axis_name="c", subcore_axis_name="s"),
    out_type=jax.ShapeDtypeStruct((_NROW, _NCOL), jnp.int32),
    scratch_types=(
        [pltpu.VMEM((_CROWS, _NCOL), jnp.float32)] * _RB
        + [pltpu.VMEM((_CROWS, _NCOL), jnp.int32)] * _RB
        + [pltpu.SemaphoreType.DMA] * (2 * _RB)
    ),
)
def _sc_bucketize(x_hbm, out_hbm, *refs):
    xv = refs[:_RB]
    ov = refs[_RB : 2 * _RB]
    si = refs[2 * _RB : 3 * _RB]
    so = refs[3 * _RB : 4 * _RB]

    wid = lax.axis_index("s") * _NC + lax.axis_index("c")
    base = wid * _ROWS_W

    for b in range(_RB):
        pltpu.async_copy(x_hbm.at[pl.ds(base + b * _CROWS, _CROWS)], xv[b], si[b])

    def turn_body(t, carry):
        row_t = base + t * _RB * _CROWS
        for b in range(_RB):
            row = row_t + b * _CROWS
            pltpu.make_async_copy(
                x_hbm.at[pl.ds(row, _CROWS)], xv[b], si[b]
            ).wait()

            @pl.when(t > 0)
            def _():
                pltpu.make_async_copy(
                    ov[b], out_hbm.at[pl.ds(row - _RB * _CROWS, _CROWS)], so[b]
                ).wait()

            _compute_chunk(xv[b], ov[b])
            pltpu.async_copy(ov[b], out_hbm.at[pl.ds(row, _CROWS)], so[b])

            @pl.when(t + 1 < _NTURN)
            def _():
                pltpu.async_copy(
                    x_hbm.at[pl.ds(row + _RB * _CROWS, _CROWS)], xv[b], si[b]
                )
        return carry

    lax.fori_loop(0, _NTURN, turn_body, 0)

    for b in range(_RB):
        row = base + _ROWS_W - (_RB - b) * _CROWS
        pltpu.make_async_copy(ov[b], out_hbm.at[pl.ds(row, _CROWS)], so[b]).wait()


def kernel(inputs):
    out = _sc_bucketize(inputs)
    return out.astype(jnp.int64)


# speedup vs baseline: 1.0057x; 1.0057x over previous
import functools

import jax
import jax.numpy as jnp
from jax import lax
from jax.experimental import pallas as pl
from jax.experimental.pallas import tpu as pltpu
from jax.experimental.pallas import tpu_sc as plsc

_NC = 2
_NS = 16
_LANES = 16
_NW = _NC * _NS

_NROW = 4096
_NCOL = 4096
_ROWS_W = _NROW // _NW
_CROWS = 2
_NCHUNK = _ROWS_W // _CROWS
_RB = 4
_NTURN = _NCHUNK // _RB


def _compute_chunk(xv, ov):
    c16 = jnp.full((_LANES,), 16, jnp.int32)
    c17 = jnp.full((_LANES,), 17, jnp.int32)
    for r in range(_CROWS):
        @plsc.parallel_loop(0, _NCOL, step=_LANES, unroll=16)
        def _(j):
            x = xv[r, pl.ds(j, _LANES)]
            y = jnp.minimum(jnp.maximum(x * 4.0, -17.0), 16.0)
            i = y.astype(jnp.int32)
            f = i.astype(jnp.float32)
            b = i + jnp.where(f > y, c16, c17)
            ov[r, pl.ds(j, _LANES)] = b


@functools.partial(
    pl.kernel,
    mesh=plsc.VectorSubcoreMesh(core_axis_name="c", subcore_axis_name="s"),
    out_type=jax.ShapeDtypeStruct((_NROW, _NCOL), jnp.int32),
    scratch_types=(
        [pltpu.VMEM((_CROWS, _NCOL), jnp.float32)] * _RB
        + [pltpu.VMEM((_CROWS, _NCOL), jnp.int32)] * _RB
        + [pltpu.SemaphoreType.DMA] * (2 * _RB)
    ),
)
def _sc_bucketize(x_hbm, out_hbm, *refs):
    xv = refs[:_RB]
    ov = refs[_RB : 2 * _RB]
    si = refs[2 * _RB : 3 * _RB]
    so = refs[3 * _RB : 4 * _RB]

    wid = lax.axis_index("s") * _NC + lax.axis_index("c")
    base = wid * _ROWS_W

    for b in range(_RB):
        pltpu.async_copy(x_hbm.at[pl.ds(base + b * _CROWS, _CROWS)], xv[b], si[b])

    def turn_body(t, carry):
        row_t = base + t * _RB * _CROWS
        for b in range(_RB):
            row = row_t + b * _CROWS
            pltpu.make_async_copy(
                x_hbm.at[pl.ds(row, _CROWS)], xv[b], si[b]
            ).wait()

            @pl.when(t > 0)
            def _():
                pltpu.make_async_copy(
                    ov[b], out_hbm.at[pl.ds(row - _RB * _CROWS, _CROWS)], so[b]
                ).wait()

            _compute_chunk(xv[b], ov[b])
            pltpu.async_copy(ov[b], out_hbm.at[pl.ds(row, _CROWS)], so[b])

            @pl.when(t + 1 < _NTURN)
            def _():
                pltpu.async_copy(
                    x_hbm.at[pl.ds(row + _RB * _CROWS, _CROWS)], xv[b], si[b]
                )
        return carry

    lax.fori_loop(0, _NTURN, turn_body, 0)

    for b in range(_RB):
        row = base + _ROWS_W - (_RB - b) * _CROWS
        pltpu.make_async_copy(ov[b], out_hbm.at[pl.ds(row, _CROWS)], so[b]).wait()


def kernel(inputs):
    out = _sc_bucketize(inputs)
    return out.astype(jnp.int64)
